# baseline (speedup 1.0000x reference)
import functools

import jax
import jax.numpy as jnp
from jax import lax
from jax.experimental import pallas as pl
from jax.experimental.pallas import tpu as pltpu
from jax.experimental.pallas import tpu_sc as plsc

NC = 2
NS = 16
L = 16
NW = NC * NS
CT = 32


def _emb_body(B, S, D, PW, ids_hbm, word_hbm, pos_hbm, type_hbm, gamma_hbm,
              beta_hbm, out_hbm, idx_v, rows0, rows1, pos_v, type_v, gamma_v,
              beta_v, gsem0, gsem1, osem0, osem1):
    J = D // L
    NCHUNK = (B * PW) // CT
    HPB = PW // CT
    wid = lax.axis_index("s") * NC + lax.axis_index("c")
    pos0 = wid * PW

    pltpu.sync_copy(pos_hbm.at[pl.ds(pos0, PW)], pos_v)
    pltpu.sync_copy(type_hbm.at[0], type_v)
    pltpu.sync_copy(gamma_hbm, gamma_v)
    pltpu.sync_copy(beta_hbm, beta_v)
    for c in range(NCHUNK):
        b, h = divmod(c, HPB)
        pltpu.sync_copy(ids_hbm.at[pl.ds(b * S + pos0 + h * CT, CT)],
                        idx_v.at[c])

    for j in range(J):
        sl = pl.ds(j * L, L)
        tv = type_v[sl]

        def pbody(p, tv):
            pos_v[p, sl] = pos_v[p, sl] + tv
            return tv

        lax.fori_loop(0, PW, pbody, tv)

    bufs = (rows0, rows1)
    gsems = (gsem0, gsem1)
    osems = (osem0, osem1)

    def chunk_slices(c):
        b, h = divmod(c, HPB)
        hbm_base = b * S + pos0 + h * CT
        return b, h, hbm_base

    def start_gather(c):
        p = c % 2
        return pltpu.async_copy(
            word_hbm.at[idx_v.at[c]], bufs[p], gsems[p])

    def compute_chunk(c):
        p = c % 2
        rows_v = bufs[p]
        _, h, _ = chunk_slices(c)

        def tbody(t, carry):
            acc_a = jnp.zeros((L,), jnp.float32)
            acc_b = jnp.zeros((L,), jnp.float32)
            acc2_a = jnp.zeros((L,), jnp.float32)
            acc2_b = jnp.zeros((L,), jnp.float32)
            for j in range(J):
                sl = pl.ds(j * L, L)
                v = rows_v[t, sl] + pos_v[h * CT + t, sl]
                rows_v[t, sl] = v
                if j % 2 == 0:
                    acc_a = acc_a + v
                    acc2_a = acc2_a + v * v
                else:
                    acc_b = acc_b + v
                    acc2_b = acc2_b + v * v
            s1 = jnp.broadcast_to(jnp.sum(acc_a + acc_b), (L,))
            s2 = jnp.broadcast_to(jnp.sum(acc2_a + acc2_b), (L,))
            mean = s1 * (1.0 / D)
            var = s2 * (1.0 / D) - mean * mean
            x = var + 1e-12
            xi = lax.bitcast_convert_type(x, jnp.int32)
            yi = jnp.int32(0x5F3759DF) - lax.shift_right_logical(xi, 1)
            y = lax.bitcast_convert_type(yi, jnp.float32)
            hx = x * 0.5
            for _ in range(3):
                y = y * (1.5 - hx * y * y)
            ms = mean * y
            for j in range(J):
                sl = pl.ds(j * L, L)
                v = rows_v[t, sl]
                rows_v[t, sl] = (v * y - ms) * gamma_v[sl] + beta_v[sl]
            return carry

        lax.fori_loop(0, CT, tbody, 0)

    def start_out(c):
        p = c % 2
        _, _, hbm_base = chunk_slices(c)
        return pltpu.async_copy(bufs[p], out_hbm.at[pl.ds(hbm_base, CT)],
                                osems[p])

    out_handles = [None, None]
    gather_handles = [None, None]
    gather_handles[0] = start_gather(0)
    for c in range(NCHUNK):
        p = c % 2
        gather_handles[p].wait()
        if c + 1 < NCHUNK:
            if out_handles[1 - p] is not None:
                out_handles[1 - p].wait()
            gather_handles[1 - p] = start_gather(c + 1)
        compute_chunk(c)
        out_handles[p] = start_out(c)
    for h in out_handles:
        if h is not None:
            h.wait()


def kernel(input_ids, word_emb, pos_emb, type_emb, ln_gamma, ln_beta):
    B, S = input_ids.shape
    V, D = word_emb.shape
    assert S % NW == 0 and D % L == 0
    PW = S // NW

    mesh = plsc.VectorSubcoreMesh(
        core_axis_name="c", subcore_axis_name="s", num_cores=NC,
        num_subcores=NS)
    fn = pl.kernel(
        functools.partial(_emb_body, B, S, D, PW),
        out_type=jax.ShapeDtypeStruct((B * S, D), jnp.float32),
        mesh=mesh,
        compiler_params=pltpu.CompilerParams(needs_layout_passes=False),
        scratch_types=[
            pltpu.VMEM(((B * PW) // CT, CT), jnp.int32),
            pltpu.VMEM((CT, D), jnp.float32),
            pltpu.VMEM((CT, D), jnp.float32),
            pltpu.VMEM((PW, D), jnp.float32),
            pltpu.VMEM((D,), jnp.float32),
            pltpu.VMEM((D,), jnp.float32),
            pltpu.VMEM((D,), jnp.float32),
            pltpu.SemaphoreType.DMA,
            pltpu.SemaphoreType.DMA,
            pltpu.SemaphoreType.DMA,
            pltpu.SemaphoreType.DMA,
        ],
    )
    out = fn(input_ids.reshape(B * S), word_emb, pos_emb, type_emb,
             ln_gamma, ln_beta)
    return out.reshape(B, S, D)
